# baseline (speedup 1.0000x reference)
import functools

import jax
import jax.numpy as jnp
from jax import lax
from jax.experimental import pallas as pl
from jax.experimental.pallas import tpu as pltpu
from jax.experimental.pallas import tpu_sc as plsc

_DIM = 128
_N = 32
_NE = 2048
_NW = 32
_EPW = _NE // _NW
_F32 = jnp.float32
_I32 = jnp.int32


def _sc_counts_body(es_hbm, ed_hbm, ef_hbm, a_out, f_out,
                    es_v, ed_v, ef_v, a_v, f_v):
    wid = lax.axis_index("s") * 2 + lax.axis_index("c")
    base = wid * _EPW
    pltpu.sync_copy(es_hbm.at[pl.ds(base, _EPW)], es_v)
    pltpu.sync_copy(ed_hbm.at[pl.ds(base, _EPW)], ed_v)
    pltpu.sync_copy(ef_hbm.at[pl.ds(base, _EPW)], ef_v)

    zeros16 = jnp.zeros((16,), _I32)
    for i in range(_N * _N // 16):
        a_v[pl.ds(i * 16, 16)] = zeros16
    for i in range(_N * 16 // 16):
        f_v[pl.ds(i * 16, 16)] = zeros16

    ones16 = jnp.ones((16,), _I32)
    for g in range(_EPW // 16):
        s16 = es_v[pl.ds(g * 16, 16)]
        d16 = ed_v[pl.ds(g * 16, 16)]
        k16 = ef_v[pl.ds(g * 16, 16)]
        ok = (s16 < _N) & (d16 < _N)
        ss = jnp.where(ok, s16, 0)
        dd = jnp.where(ok, d16, 0)
        plsc.addupdate_scatter(a_v, [dd * _N + ss], ones16, mask=ok)
        plsc.addupdate_scatter(f_v, [dd * 16 + k16], ones16, mask=ok)

    pltpu.sync_copy(a_v, a_out.at[wid])
    pltpu.sync_copy(f_v, f_out.at[wid])


_sc_counts = functools.partial(
    pl.kernel,
    mesh=plsc.VectorSubcoreMesh(core_axis_name="c", subcore_axis_name="s"),
    compiler_params=pltpu.CompilerParams(needs_layout_passes=False),
    out_type=[
        jax.ShapeDtypeStruct((_NW, _N * _N), _I32),
        jax.ShapeDtypeStruct((_NW, _N * 16), _I32),
    ],
    scratch_types=[
        pltpu.VMEM((_EPW,), _I32),
        pltpu.VMEM((_EPW,), _I32),
        pltpu.VMEM((_EPW,), _I32),
        pltpu.VMEM((_N * _N,), _I32),
        pltpu.VMEM((_N * 16,), _I32),
    ],
)(_sc_counts_body)


def _tc_body(a_ref, f_ref, nt_ref, tr_ref,
             ne_w_ref, te_w_ref, efw_ref,
             w_ih_ref, w_hh_ref, b_ih_ref, b_hh_ref, ng_ref, nb_ref,
             W1_ref, b1_ref, g2_ref, bt2_ref, W2_ref, b2_ref, out_ref):
    A = jnp.sum(a_ref[...], axis=0).astype(_F32)
    F = jnp.sum(f_ref[...], axis=0).astype(_F32)
    E = jnp.dot(F, efw_ref[:], preferred_element_type=_F32)
    cnt = jnp.sum(A, axis=1, keepdims=True)
    inv_cnt = 1.0 / jnp.maximum(cnt, 1.0)

    nt_c = nt_ref[:]
    tr_c = tr_ref[:]
    oh_nt = (nt_c == lax.broadcasted_iota(_I32, (_N, 20), 1)).astype(_F32)
    oh_tr = (tr_c == lax.broadcasted_iota(_I32, (_N, 6), 1)).astype(_F32)
    h = (jnp.dot(oh_nt, ne_w_ref[:], preferred_element_type=_F32)
         + jnp.dot(oh_tr, te_w_ref[:], preferred_element_type=_F32))

    w_ih = w_ih_ref[:]
    w_hh = w_hh_ref[:]
    b_ih = b_ih_ref[:]
    b_hh = b_hh_ref[:]
    ng = ng_ref[:]
    nb = nb_ref[:]
    nt_dims = (((1,), (1,)), ((), ()))

    for _ in range(5):
        agg = (jnp.dot(A, h, preferred_element_type=_F32) + E) * inv_cnt
        gi = lax.dot_general(agg, w_ih, nt_dims,
                             preferred_element_type=_F32) + b_ih
        gh = lax.dot_general(h, w_hh, nt_dims,
                             preferred_element_type=_F32) + b_hh
        r = jax.nn.sigmoid(gi[:, 0:128] + gh[:, 0:128])
        z = jax.nn.sigmoid(gi[:, 128:256] + gh[:, 128:256])
        n = jnp.tanh(gi[:, 256:384] + r * gh[:, 256:384])
        hn = (1.0 - z) * n + z * h
        mu = jnp.mean(hn, axis=1, keepdims=True)
        var = jnp.mean((hn - mu) ** 2, axis=1, keepdims=True)
        h = (hn - mu) / jnp.sqrt(var + 1e-5) * ng + nb

    hmean = jnp.mean(h, axis=0, keepdims=True)
    hmax = jnp.max(h, axis=0, keepdims=True)
    pooled = jnp.concatenate([hmean, hmax], axis=1)
    x = lax.dot_general(pooled, W1_ref[:], nt_dims,
                        preferred_element_type=_F32) + b1_ref[:]
    mu = jnp.mean(x, axis=1, keepdims=True)
    var = jnp.mean((x - mu) ** 2, axis=1, keepdims=True)
    x = (x - mu) / jnp.sqrt(var + 1e-5) * g2_ref[:] + bt2_ref[:]
    x = jnp.maximum(x, 0.0)
    out_row = x * W2_ref[:] + b2_ref[:] * (1.0 / 128.0)
    out_ref[:, :] = jnp.sum(out_row, axis=1, keepdims=True)


def kernel(nt, tr, es, ed, ef, ne_w, te_w, ef_w, w_ih, w_hh, b_ih, b_hh,
           ng, nb, W1, b1, g2, bt2, W2, b2):
    a_part, f_part = _sc_counts(
        es.astype(_I32), ed.astype(_I32), ef.astype(_I32))
    efw_pad = jnp.zeros((16, _DIM), _F32).at[0:6, :].set(ef_w)
    out = pl.pallas_call(
        _tc_body,
        out_shape=jax.ShapeDtypeStruct((1, 1), _F32),
    )(
        a_part.reshape(_NW, _N, _N), f_part.reshape(_NW, _N, 16),
        nt.reshape(_N, 1).astype(_I32), tr.reshape(_N, 1).astype(_I32),
        ne_w, te_w, efw_pad,
        w_ih, w_hh, b_ih.reshape(1, 3 * _DIM), b_hh.reshape(1, 3 * _DIM),
        ng.reshape(1, _DIM), nb.reshape(1, _DIM),
        W1, b1.reshape(1, _DIM), g2.reshape(1, _DIM), bt2.reshape(1, _DIM),
        W2, jnp.broadcast_to(b2.reshape(1, 1), (1, _DIM)),
    )
    return out.reshape(())


# speedup vs baseline: 1.0779x; 1.0779x over previous
import functools

import jax
import jax.numpy as jnp
from jax import lax
from jax.experimental import pallas as pl
from jax.experimental.pallas import tpu as pltpu
from jax.experimental.pallas import tpu_sc as plsc

_DIM = 128
_N = 32
_NE = 2048
_NW = 16
_EPW = _NE // _NW
_F32 = jnp.float32
_I32 = jnp.int32


def _sc_counts_body(es_hbm, ed_hbm, ef_hbm, a_out, f_out,
                    es_v, ed_v, ef_v, a_v, f_v):
    wid = lax.axis_index("s")
    base = wid * _EPW
    pltpu.sync_copy(es_hbm.at[pl.ds(base, _EPW)], es_v)
    pltpu.sync_copy(ed_hbm.at[pl.ds(base, _EPW)], ed_v)
    pltpu.sync_copy(ef_hbm.at[pl.ds(base, _EPW)], ef_v)

    zeros16 = jnp.zeros((16,), _I32)
    for i in range(_N * _N // 16):
        a_v[pl.ds(i * 16, 16)] = zeros16
    for i in range(_N * 16 // 16):
        f_v[pl.ds(i * 16, 16)] = zeros16

    ones16 = jnp.ones((16,), _I32)
    for g in range(_EPW // 16):
        s16 = es_v[pl.ds(g * 16, 16)]
        d16 = ed_v[pl.ds(g * 16, 16)]
        k16 = ef_v[pl.ds(g * 16, 16)]
        ok = (s16 < _N) & (d16 < _N)
        ss = jnp.where(ok, s16, 0)
        dd = jnp.where(ok, d16, 0)
        plsc.addupdate_scatter(a_v, [dd * _N + ss], ones16, mask=ok)
        plsc.addupdate_scatter(f_v, [dd * 16 + k16], ones16, mask=ok)

    pltpu.sync_copy(a_v, a_out.at[wid])
    pltpu.sync_copy(f_v, f_out.at[wid])


_sc_counts = functools.partial(
    pl.kernel,
    mesh=plsc.VectorSubcoreMesh(core_axis_name="c", subcore_axis_name="s",
                                num_cores=1),
    compiler_params=pltpu.CompilerParams(needs_layout_passes=False),
    out_type=[
        jax.ShapeDtypeStruct((_NW, _N * _N), _I32),
        jax.ShapeDtypeStruct((_NW, _N * 16), _I32),
    ],
    scratch_types=[
        pltpu.VMEM((_EPW,), _I32),
        pltpu.VMEM((_EPW,), _I32),
        pltpu.VMEM((_EPW,), _I32),
        pltpu.VMEM((_N * _N,), _I32),
        pltpu.VMEM((_N * 16,), _I32),
    ],
)(_sc_counts_body)


def _tc_body(a_ref, f_ref, nt_ref, tr_ref,
             ne_w_ref, te_w_ref, efw_ref,
             w_ih_ref, w_hh_ref, b_ih_ref, b_hh_ref, ng_ref, nb_ref,
             W1_ref, b1_ref, g2_ref, bt2_ref, W2_ref, b2_ref, out_ref):
    A = jnp.sum(a_ref[...], axis=0).astype(_F32)
    F = jnp.sum(f_ref[...], axis=0).astype(_F32)
    E = jnp.dot(F, efw_ref[:], preferred_element_type=_F32)
    cnt = jnp.sum(A, axis=1, keepdims=True)
    inv_cnt = 1.0 / jnp.maximum(cnt, 1.0)

    nt_c = nt_ref[:]
    tr_c = tr_ref[:]
    oh_nt = (nt_c == lax.broadcasted_iota(_I32, (_N, 20), 1)).astype(_F32)
    oh_tr = (tr_c == lax.broadcasted_iota(_I32, (_N, 6), 1)).astype(_F32)
    h = (jnp.dot(oh_nt, ne_w_ref[:], preferred_element_type=_F32)
         + jnp.dot(oh_tr, te_w_ref[:], preferred_element_type=_F32))

    w_ih = w_ih_ref[:]
    w_hh = w_hh_ref[:]
    b_ih = b_ih_ref[:]
    b_hh = b_hh_ref[:]
    ng = ng_ref[:]
    nb = nb_ref[:]
    nt_dims = (((1,), (1,)), ((), ()))

    for _ in range(5):
        agg = (jnp.dot(A, h, preferred_element_type=_F32) + E) * inv_cnt
        gi = lax.dot_general(agg, w_ih, nt_dims,
                             preferred_element_type=_F32) + b_ih
        gh = lax.dot_general(h, w_hh, nt_dims,
                             preferred_element_type=_F32) + b_hh
        r = jax.nn.sigmoid(gi[:, 0:128] + gh[:, 0:128])
        z = jax.nn.sigmoid(gi[:, 128:256] + gh[:, 128:256])
        n = jnp.tanh(gi[:, 256:384] + r * gh[:, 256:384])
        hn = (1.0 - z) * n + z * h
        mu = jnp.mean(hn, axis=1, keepdims=True)
        var = jnp.mean((hn - mu) ** 2, axis=1, keepdims=True)
        h = (hn - mu) / jnp.sqrt(var + 1e-5) * ng + nb

    hmean = jnp.mean(h, axis=0, keepdims=True)
    hmax = jnp.max(h, axis=0, keepdims=True)
    pooled = jnp.concatenate([hmean, hmax], axis=1)
    x = lax.dot_general(pooled, W1_ref[:], nt_dims,
                        preferred_element_type=_F32) + b1_ref[:]
    mu = jnp.mean(x, axis=1, keepdims=True)
    var = jnp.mean((x - mu) ** 2, axis=1, keepdims=True)
    x = (x - mu) / jnp.sqrt(var + 1e-5) * g2_ref[:] + bt2_ref[:]
    x = jnp.maximum(x, 0.0)
    out_row = x * W2_ref[:] + b2_ref[:] * (1.0 / 128.0)
    out_ref[:, :] = jnp.sum(out_row, axis=1, keepdims=True)


def kernel(nt, tr, es, ed, ef, ne_w, te_w, ef_w, w_ih, w_hh, b_ih, b_hh,
           ng, nb, W1, b1, g2, bt2, W2, b2):
    a_part, f_part = _sc_counts(
        es.astype(_I32), ed.astype(_I32), ef.astype(_I32))
    efw_pad = jnp.zeros((16, _DIM), _F32).at[0:6, :].set(ef_w)
    out = pl.pallas_call(
        _tc_body,
        out_shape=jax.ShapeDtypeStruct((1, 1), _F32),
    )(
        a_part.reshape(_NW, _N, _N), f_part.reshape(_NW, _N, 16),
        nt.reshape(_N, 1).astype(_I32), tr.reshape(_N, 1).astype(_I32),
        ne_w, te_w, efw_pad,
        w_ih, w_hh, b_ih.reshape(1, 3 * _DIM), b_hh.reshape(1, 3 * _DIM),
        ng.reshape(1, _DIM), nb.reshape(1, _DIM),
        W1, b1.reshape(1, _DIM), g2.reshape(1, _DIM), bt2.reshape(1, _DIM),
        W2, jnp.broadcast_to(b2.reshape(1, 1), (1, _DIM)),
    )
    return out.reshape(())


# speedup vs baseline: 1.2045x; 1.1174x over previous
import functools

import jax
import jax.numpy as jnp
from jax import lax
from jax.experimental import pallas as pl
from jax.experimental.pallas import tpu as pltpu
from jax.experimental.pallas import tpu_sc as plsc

_DIM = 128
_N = 32
_NE = 2048
_NW = 16
_EPW = _NE // _NW
_F32 = jnp.float32
_I32 = jnp.int32


def _sc_counts_body(es_hbm, ed_hbm, ef_hbm, a_out, f_out,
                    es_v, ed_v, ef_v, a_v, f_v, sem):
    wid = lax.axis_index("s")
    base = wid * _EPW
    c1 = pltpu.async_copy(es_hbm.at[pl.ds(base, _EPW)], es_v, sem)
    c2 = pltpu.async_copy(ed_hbm.at[pl.ds(base, _EPW)], ed_v, sem)
    c3 = pltpu.async_copy(ef_hbm.at[pl.ds(base, _EPW)], ef_v, sem)

    zeros16 = jnp.zeros((16,), _I32)
    for r in range(_N):
        a_v[r, pl.ds(0, 16)] = zeros16
        a_v[r, pl.ds(16, 16)] = zeros16
        f_v[r, pl.ds(0, 16)] = zeros16
    c1.wait()
    c2.wait()
    c3.wait()

    ones16 = jnp.ones((16,), _I32)
    for g in range(_EPW // 16):
        s16 = es_v[pl.ds(g * 16, 16)]
        d16 = ed_v[pl.ds(g * 16, 16)]
        k16 = ef_v[pl.ds(g * 16, 16)]
        ok = (s16 < _N) & (d16 < _N)
        ss = jnp.where(ok, s16, 0)
        dd = jnp.where(ok, d16, 0)
        plsc.addupdate_scatter(a_v, [dd, ss], ones16, mask=ok)
        plsc.addupdate_scatter(f_v, [dd, k16], ones16, mask=ok)

    o1 = pltpu.async_copy(a_v, a_out.at[wid], sem)
    o2 = pltpu.async_copy(f_v, f_out.at[wid], sem)
    o1.wait()
    o2.wait()


_sc_counts = functools.partial(
    pl.kernel,
    mesh=plsc.VectorSubcoreMesh(core_axis_name="c", subcore_axis_name="s",
                                num_cores=1),
    compiler_params=pltpu.CompilerParams(needs_layout_passes=False),
    out_type=[
        jax.ShapeDtypeStruct((_NW, _N, _N), _I32),
        jax.ShapeDtypeStruct((_NW, _N, 16), _I32),
    ],
    scratch_types=[
        pltpu.VMEM((_EPW,), _I32),
        pltpu.VMEM((_EPW,), _I32),
        pltpu.VMEM((_EPW,), _I32),
        pltpu.VMEM((_N, _N), _I32),
        pltpu.VMEM((_N, 16), _I32),
        pltpu.SemaphoreType.DMA,
    ],
)(_sc_counts_body)


def _tc_body(a_ref, f_ref, nt_ref, tr_ref,
             ne_w_ref, te_w_ref, efw_ref,
             w_ih_ref, w_hh_ref, b_ih_ref, b_hh_ref, ng_ref, nb_ref,
             W1_ref, b1_ref, g2_ref, bt2_ref, W2_ref, b2_ref, out_ref):
    A = jnp.sum(a_ref[...], axis=0).astype(_F32)
    F = jnp.sum(f_ref[...], axis=0).astype(_F32)
    E = jnp.dot(F, efw_ref[:], preferred_element_type=_F32)
    cnt = jnp.sum(A, axis=1, keepdims=True)
    inv_cnt = 1.0 / jnp.maximum(cnt, 1.0)

    nt_c = nt_ref[:]
    tr_c = tr_ref[:]
    oh_nt = (nt_c == lax.broadcasted_iota(_I32, (_N, 20), 1)).astype(_F32)
    oh_tr = (tr_c == lax.broadcasted_iota(_I32, (_N, 6), 1)).astype(_F32)
    h = (jnp.dot(oh_nt, ne_w_ref[:], preferred_element_type=_F32)
         + jnp.dot(oh_tr, te_w_ref[:], preferred_element_type=_F32))

    w_ih = w_ih_ref[:]
    w_hh = w_hh_ref[:]
    b_ih = b_ih_ref[:]
    b_hh = b_hh_ref[:]
    ng = ng_ref[:]
    nb = nb_ref[:]
    nt_dims = (((1,), (1,)), ((), ()))

    for _ in range(5):
        agg = (jnp.dot(A, h, preferred_element_type=_F32) + E) * inv_cnt
        gi = lax.dot_general(agg, w_ih, nt_dims,
                             preferred_element_type=_F32) + b_ih
        gh = lax.dot_general(h, w_hh, nt_dims,
                             preferred_element_type=_F32) + b_hh
        r = jax.nn.sigmoid(gi[:, 0:128] + gh[:, 0:128])
        z = jax.nn.sigmoid(gi[:, 128:256] + gh[:, 128:256])
        n = jnp.tanh(gi[:, 256:384] + r * gh[:, 256:384])
        hn = (1.0 - z) * n + z * h
        mu = jnp.mean(hn, axis=1, keepdims=True)
        var = jnp.mean((hn - mu) ** 2, axis=1, keepdims=True)
        h = (hn - mu) / jnp.sqrt(var + 1e-5) * ng + nb

    hmean = jnp.mean(h, axis=0, keepdims=True)
    hmax = jnp.max(h, axis=0, keepdims=True)
    pooled = jnp.concatenate([hmean, hmax], axis=1)
    x = lax.dot_general(pooled, W1_ref[:], nt_dims,
                        preferred_element_type=_F32) + b1_ref[:]
    mu = jnp.mean(x, axis=1, keepdims=True)
    var = jnp.mean((x - mu) ** 2, axis=1, keepdims=True)
    x = (x - mu) / jnp.sqrt(var + 1e-5) * g2_ref[:] + bt2_ref[:]
    x = jnp.maximum(x, 0.0)
    out_row = x * W2_ref[:] + b2_ref[:] * (1.0 / 128.0)
    out_ref[:, :] = jnp.sum(out_row, axis=1, keepdims=True)


def kernel(nt, tr, es, ed, ef, ne_w, te_w, ef_w, w_ih, w_hh, b_ih, b_hh,
           ng, nb, W1, b1, g2, bt2, W2, b2):
    a_part, f_part = _sc_counts(
        es.astype(_I32), ed.astype(_I32), ef.astype(_I32))
    efw_pad = jnp.zeros((16, _DIM), _F32).at[0:6, :].set(ef_w)
    out = pl.pallas_call(
        _tc_body,
        out_shape=jax.ShapeDtypeStruct((1, 1), _F32),
    )(
        a_part, f_part,
        nt.reshape(_N, 1).astype(_I32), tr.reshape(_N, 1).astype(_I32),
        ne_w, te_w, efw_pad,
        w_ih, w_hh, b_ih.reshape(1, 3 * _DIM), b_hh.reshape(1, 3 * _DIM),
        ng.reshape(1, _DIM), nb.reshape(1, _DIM),
        W1, b1.reshape(1, _DIM), g2.reshape(1, _DIM), bt2.reshape(1, _DIM),
        W2, jnp.broadcast_to(b2.reshape(1, 1), (1, _DIM)),
    )
    return out.reshape(())
